# baseline (speedup 1.0000x reference)
import functools

import jax
import jax.numpy as jnp
from jax import lax
from jax.experimental import pallas as pl
from jax.experimental.pallas import tpu as pltpu
from jax.experimental.pallas import tpu_sc as plsc

B = 16384
F = 26
K = 16
NN = 49

NC = 2
NS = 16
NW = NC * NS
ROWS_W = B // NW
CB = 64
CBF = CB * F
NCHUNK = ROWS_W // CB
IDXR = CBF // 128
NG = CB // 16
LR = (16 - 1) * F + 1
LL = LR


def _fm_body(xcat_hbm, xnumt_hbm, lin_hbm, v_hbm, w_hbm, out_hbm,
             idx_v, vrows, linrows, xnumt_v, w_v, out_v,
             sem_v0, sem_v1, sem_l0, sem_l1, sem_x0, sem_x1):
    wid = lax.axis_index("s") * NC + lax.axis_index("c")
    sem_v = (sem_v0, sem_v1)
    sem_l = (sem_l0, sem_l1)
    sem_x = (sem_x0, sem_x1)
    pltpu.sync_copy(w_hbm, w_v)
    iota = lax.iota(jnp.int32, 16)
    iota26 = iota * F
    wchunks = [w_v[pl.ds(c * 16, 16)] for c in range(4)]
    ws = [wchunks[j // 16][j % 16] for j in range(NN)]
    bconst = wchunks[NN // 16][NN % 16]
    zeros = jnp.zeros((16,), jnp.float32)

    def issue(c, bb):
        b0 = wid * ROWS_W + c * CB
        pltpu.sync_copy(xcat_hbm.at[pl.ds(b0 * F, CBF)], idx_v.at[bb])
        for i in range(IDXR):
            sl = pl.ds(i * 128, 128)
            pltpu.async_copy(v_hbm.at[idx_v.at[bb, sl]], vrows.at[bb, sl],
                             sem_v[bb])
            pltpu.async_copy(lin_hbm.at[idx_v.at[bb, sl]], linrows.at[bb, sl],
                             sem_l[bb])
        pltpu.async_copy(xnumt_hbm.at[:, pl.ds(b0, CB)], xnumt_v.at[bb],
                         sem_x[bb])

    def compute(c, bb):
        pltpu.make_async_copy(v_hbm.at[pl.ds(0, CBF)], vrows.at[bb],
                              sem_v[bb]).wait()
        pltpu.make_async_copy(lin_hbm.at[pl.ds(0, CBF)], linrows.at[bb],
                              sem_l[bb]).wait()
        pltpu.make_async_copy(xnumt_hbm.at[:, pl.ds(0, CB)], xnumt_v.at[bb],
                              sem_x[bb]).wait()
        lref = linrows.at[bb]

        lints = []
        for g in range(NG):
            lint = zeros
            for f in range(F):
                off = g * 16 * F + f
                lint = lint + plsc.load_gather(lref, [iota26 + off])
            lints.append(lint)

        def kbody(k, carry):
            soss, acc2s = carry
            kk = jnp.full((16,), k, jnp.int32)
            new_soss, new_acc2s = [], []
            for g in range(NG):
                acc = zeros
                a2 = acc2s[g]
                for f in range(F):
                    off = g * 16 * F + f
                    val = plsc.load_gather(
                        vrows.at[bb, pl.ds(off, LR)], [iota26, kk])
                    acc = acc + val
                    a2 = a2 + val * val
                new_soss.append(soss[g] + acc * acc)
                new_acc2s.append(a2)
            return (tuple(new_soss), tuple(new_acc2s))

        z4 = (zeros,) * NG
        soss, acc2s = lax.fori_loop(0, K, kbody, (z4, z4))

        for g in range(NG):
            tot = lints[g] + 0.5 * (soss[g] - acc2s[g]) + bconst
            for j in range(NN):
                xv = xnumt_v[bb, j, pl.ds(g * 16, 16)]
                tot = tot + ws[j] * xv
            out_v[pl.ds(c * CB + g * 16, 16)] = tot

    issue(0, 0)

    def pair(p, carry):
        c0 = p * 2
        issue(c0 + 1, 1)
        compute(c0, 0)

        @pl.when(p < (NCHUNK // 2 - 1))
        def _():
            issue(c0 + 2, 0)

        compute(c0 + 1, 1)
        return carry

    lax.fori_loop(0, NCHUNK // 2, pair, 0)
    pltpu.sync_copy(out_v, out_hbm.at[pl.ds(wid * ROWS_W, ROWS_W)])


@functools.partial(jax.jit, static_argnames=())
def kernel(x_cat, x_num, lin_table, v_table, W_num, b_num, bias):
    xcat_flat = x_cat.reshape(B * F)
    xnum_t = x_num.T
    wvec = jnp.concatenate([
        W_num.reshape(-1),
        (b_num + bias).reshape(-1),
        jnp.zeros((14,), jnp.float32),
    ])
    mesh = plsc.VectorSubcoreMesh(core_axis_name="c", subcore_axis_name="s",
                                  num_cores=NC, num_subcores=NS)
    out = pl.kernel(
        _fm_body,
        out_type=jax.ShapeDtypeStruct((B,), jnp.float32),
        mesh=mesh,
        compiler_params=pltpu.CompilerParams(needs_layout_passes=False,
                                             use_tc_tiling_on_sc=False),
        scratch_types=[
            pltpu.VMEM((2, CBF), jnp.int32),
            pltpu.VMEM((2, CBF, K), jnp.float32),
            pltpu.VMEM((2, CBF), jnp.float32),
            pltpu.VMEM((2, NN, CB), jnp.float32),
            pltpu.VMEM((64,), jnp.float32),
            pltpu.VMEM((ROWS_W,), jnp.float32),
            pltpu.SemaphoreType.DMA,
            pltpu.SemaphoreType.DMA,
            pltpu.SemaphoreType.DMA,
            pltpu.SemaphoreType.DMA,
            pltpu.SemaphoreType.DMA,
            pltpu.SemaphoreType.DMA,
        ],
    )(xcat_flat, xnum_t, lin_table.reshape(-1), v_table, wvec)
    return out.reshape(B, 1)
